# baseline (speedup 1.0000x reference)
import functools

import jax
import jax.numpy as jnp
from jax.experimental import pallas as pl
from jax.experimental.pallas import tpu as pltpu

D_MODEL = 1024
N_HEADS = 16
DPH = D_MODEL // N_HEADS
QH = 4
DPQ = D_MODEL // QH
K_CODES = 1024
S_CHUNK = 1024


def _fused(enc_ref, wk_ref, wv_ref, bv_ref, cb_ref,
           out_ref, idx_ref, loss_ref,
           pall_ref, *, loss_scale, n_batch, seq_len):
    b = pl.program_id(0)
    nc = seq_len // S_CHUNK
    wk = wk_ref[...]

    parts = []
    for c in range(nc):
        ec = enc_ref[0, pl.ds(c * S_CHUNK, S_CHUNK), :]
        st = jax.lax.dot_general(wk, ec, (((0,), (1,)), ((), ())),
                                 preferred_element_type=jnp.float32)
        mc = jnp.max(st, axis=1, keepdims=True)
        p = jnp.exp(st - mc)
        lc = jnp.sum(p, axis=1, keepdims=True)
        ac = jax.lax.dot_general(p, ec, (((1,), (0,)), ((), ())),
                                 preferred_element_type=jnp.float32)
        parts.append((mc, lc, ac))

    m = parts[0][0]
    for c in range(1, nc):
        m = jnp.maximum(m, parts[c][0])
    l = jnp.zeros_like(parts[0][1])
    acc = jnp.zeros_like(parts[0][2])
    for mc, lc, ac in parts:
        w = jnp.exp(mc - m)
        l = l + lc * w
        acc = acc + ac * w
    pooled = acc / l
    proj = jax.lax.dot_general(pooled, wv_ref[...], (((1,), (0,)), ((), ())),
                               preferred_element_type=jnp.float32)
    row = jax.lax.broadcasted_iota(jnp.int32, (N_HEADS, D_MODEL), 0)
    col = jax.lax.broadcasted_iota(jnp.int32, (N_HEADS, D_MODEL), 1)
    mask = (col // DPH == row).astype(jnp.float32)
    zb = jnp.sum(proj * mask, axis=0, keepdims=True) + bv_ref[...]
    pall_ref[pl.ds(b, 1), :] = zb

    @pl.when(b == n_batch - 1)
    def _finalize():
        zall = pall_ref[...]
        iota_k = jax.lax.broadcasted_iota(jnp.int32, (n_batch, K_CODES), 1)
        ssq = jnp.float32(0.0)
        idx_cols = []
        for h in range(QH):
            cbh = cb_ref[h]
            zh = zall[:, h * DPQ:(h + 1) * DPQ]
            dots = jax.lax.dot_general(zh, cbh, (((1,), (1,)), ((), ())),
                                       preferred_element_type=jnp.float32)
            csq = jax.lax.dot_general(jnp.ones((1, DPQ), jnp.float32), cbh * cbh,
                                      (((1,), (1,)), ((), ())),
                                      preferred_element_type=jnp.float32)
            zsq = jnp.sum(zh * zh, axis=1, keepdims=True)
            dist = zsq + csq - 2.0 * dots
            md = jnp.min(dist, axis=1, keepdims=True)
            idxs = jnp.min(jnp.where(dist == md, iota_k, K_CODES),
                           axis=1, keepdims=True)
            idx_cols.append(idxs)
            onehot = (iota_k == idxs).astype(jnp.float32)
            q = jax.lax.dot_general(onehot, cbh, (((1,), (0,)), ((), ())),
                                    preferred_element_type=jnp.float32)
            out_ref[:, 0, h * DPQ:(h + 1) * DPQ] = q
            d = q - zh
            ssq = ssq + jnp.sum(d * d)

        idx_ref[...] = jnp.concatenate(idx_cols, axis=1)
        loss_ref[0, 0] = ssq * loss_scale


def kernel(encoding, W_k, b_k, W_v, b_v, codebook, global_step):
    del b_k, global_step
    B, S, D = encoding.shape
    bv = b_v.reshape(1, D)
    body = functools.partial(_fused, loss_scale=0.25 / (B * QH * DPQ),
                             n_batch=B, seq_len=S)
    out, idx, loss = pl.pallas_call(
        body,
        grid=(B,),
        in_specs=[
            pl.BlockSpec((1, S, D), lambda b: (b, 0, 0)),
            pl.BlockSpec((D, N_HEADS), lambda b: (0, 0)),
            pl.BlockSpec((D, D), lambda b: (0, 0)),
            pl.BlockSpec((1, D), lambda b: (0, 0)),
            pl.BlockSpec((QH, K_CODES, DPQ), lambda b: (0, 0, 0)),
        ],
        out_specs=[
            pl.BlockSpec((B, 1, D), lambda b: (0, 0, 0)),
            pl.BlockSpec((B, QH), lambda b: (0, 0)),
            pl.BlockSpec(memory_space=pltpu.SMEM),
        ],
        out_shape=[
            jax.ShapeDtypeStruct((B, 1, D), jnp.float32),
            jax.ShapeDtypeStruct((B, QH), jnp.int32),
            jax.ShapeDtypeStruct((1, 1), jnp.float32),
        ],
        scratch_shapes=[
            pltpu.VMEM((B, D), jnp.float32),
        ],
    )(encoding, W_k, W_v, bv, codebook)
    return out, loss.reshape(()), idx


# speedup vs baseline: 1.0214x; 1.0214x over previous
import functools

import jax
import jax.numpy as jnp
from jax.experimental import pallas as pl
from jax.experimental.pallas import tpu as pltpu

D_MODEL = 1024
N_HEADS = 16
DPH = D_MODEL // N_HEADS
QH = 4
DPQ = D_MODEL // QH
K_CODES = 1024


def _fused(enc0_ref, enc1_ref, wk_ref, wv_ref, bv_ref, cb_ref,
           out_ref, idx_ref, loss_ref,
           pall_ref, *, loss_scale, n_batch):
    b = pl.program_id(0)
    wk = wk_ref[...]

    parts = []
    for ref in (enc0_ref, enc1_ref):
        ec = ref[0]
        st = jax.lax.dot_general(wk, ec, (((0,), (1,)), ((), ())),
                                 preferred_element_type=jnp.float32)
        mc = jnp.max(st, axis=1, keepdims=True)
        p = jnp.exp(st - mc)
        lc = jnp.sum(p, axis=1, keepdims=True)
        ac = jax.lax.dot_general(p, ec, (((1,), (0,)), ((), ())),
                                 preferred_element_type=jnp.float32)
        parts.append((mc, lc, ac))

    (m0, l0, a0), (m1, l1, a1) = parts
    m = jnp.maximum(m0, m1)
    w0 = jnp.exp(m0 - m)
    w1 = jnp.exp(m1 - m)
    l = l0 * w0 + l1 * w1
    acc = a0 * w0 + a1 * w1
    pall_ref[pl.ds(N_HEADS * b, N_HEADS), :] = acc / l

    @pl.when(b == n_batch - 1)
    def _finalize():
        BH = n_batch * N_HEADS
        proj = jax.lax.dot_general(pall_ref[...], wv_ref[...],
                                   (((1,), (0,)), ((), ())),
                                   preferred_element_type=jnp.float32)
        row = jax.lax.broadcasted_iota(jnp.int32, (BH, D_MODEL), 0)
        col = jax.lax.broadcasted_iota(jnp.int32, (BH, D_MODEL), 1)
        mask = (col // DPH == row % N_HEADS).astype(jnp.float32)
        zall = (jnp.sum((proj * mask).reshape(n_batch, N_HEADS, D_MODEL), axis=1)
                + bv_ref[...])

        iota_k = jax.lax.broadcasted_iota(jnp.int32, (n_batch, K_CODES), 1)
        ssq = jnp.float32(0.0)
        idx_cols = []
        for h in range(QH):
            cbh = cb_ref[h]
            zh = zall[:, h * DPQ:(h + 1) * DPQ]
            dots = jax.lax.dot_general(zh, cbh, (((1,), (1,)), ((), ())),
                                       preferred_element_type=jnp.float32)
            csq = jax.lax.dot_general(jnp.ones((1, DPQ), jnp.float32), cbh * cbh,
                                      (((1,), (1,)), ((), ())),
                                      preferred_element_type=jnp.float32)
            zsq = jnp.sum(zh * zh, axis=1, keepdims=True)
            dist = zsq + csq - 2.0 * dots
            md = jnp.min(dist, axis=1, keepdims=True)
            idxs = jnp.min(jnp.where(dist == md, iota_k, K_CODES),
                           axis=1, keepdims=True)
            idx_cols.append(idxs)
            onehot = (iota_k == idxs).astype(jnp.float32)
            q = jax.lax.dot_general(onehot, cbh, (((1,), (0,)), ((), ())),
                                    preferred_element_type=jnp.float32)
            out_ref[:, 0, h * DPQ:(h + 1) * DPQ] = q
            d = q - zh
            ssq = ssq + jnp.sum(d * d)

        idx_ref[...] = jnp.concatenate(idx_cols, axis=1)
        loss_ref[0, 0] = ssq * loss_scale


def kernel(encoding, W_k, b_k, W_v, b_v, codebook, global_step):
    del b_k, global_step
    B, S, D = encoding.shape
    half = S // 2
    bv = b_v.reshape(1, D)
    body = functools.partial(_fused, loss_scale=0.25 / (B * QH * DPQ), n_batch=B)
    out, idx, loss = pl.pallas_call(
        body,
        grid=(B,),
        in_specs=[
            pl.BlockSpec((1, half, D), lambda b: (b, 0, 0)),
            pl.BlockSpec((1, half, D), lambda b: (b, 1, 0)),
            pl.BlockSpec((D, N_HEADS), lambda b: (0, 0)),
            pl.BlockSpec((D, D), lambda b: (0, 0)),
            pl.BlockSpec((1, D), lambda b: (0, 0)),
            pl.BlockSpec((QH, K_CODES, DPQ), lambda b: (0, 0, 0)),
        ],
        out_specs=[
            pl.BlockSpec((B, 1, D), lambda b: (0, 0, 0)),
            pl.BlockSpec((B, QH), lambda b: (0, 0)),
            pl.BlockSpec(memory_space=pltpu.SMEM),
        ],
        out_shape=[
            jax.ShapeDtypeStruct((B, 1, D), jnp.float32),
            jax.ShapeDtypeStruct((B, QH), jnp.int32),
            jax.ShapeDtypeStruct((1, 1), jnp.float32),
        ],
        scratch_shapes=[
            pltpu.VMEM((B * N_HEADS, D), jnp.float32),
        ],
    )(encoding, encoding, W_k, W_v, bv, codebook)
    return out, loss.reshape(()), idx


# speedup vs baseline: 1.0247x; 1.0033x over previous
import functools

import jax
import jax.numpy as jnp
from jax.experimental import pallas as pl
from jax.experimental.pallas import tpu as pltpu

D_MODEL = 1024
N_HEADS = 16
DPH = D_MODEL // N_HEADS
QH = 4
DPQ = D_MODEL // QH
K_CODES = 1024
S_BLK = 4096


def _fused(enc_ref, wk_ref, wv_ref, bv_ref, cb_ref,
           out_ref, idx_ref, loss_ref,
           m_ref, l_ref, acc_ref, pall_ref, *, loss_scale, n_batch):
    b = pl.program_id(0)
    s = pl.program_id(1)
    ns = pl.num_programs(1)

    @pl.when(s == 0)
    def _init():
        m_ref[...] = jnp.full_like(m_ref, -jnp.inf)
        l_ref[...] = jnp.zeros_like(l_ref)
        acc_ref[...] = jnp.zeros_like(acc_ref)

    enc = enc_ref[0]
    st = jax.lax.dot_general(wk_ref[...], enc, (((0,), (1,)), ((), ())),
                             preferred_element_type=jnp.float32)
    m_old = m_ref[...]
    m_new = jnp.maximum(m_old, jnp.max(st, axis=1, keepdims=True))
    corr = jnp.exp(m_old - m_new)
    p = jnp.exp(st - m_new)
    l_ref[...] = l_ref[...] * corr + jnp.sum(p, axis=1, keepdims=True)
    pe = jax.lax.dot_general(p, enc, (((1,), (0,)), ((), ())),
                             preferred_element_type=jnp.float32)
    acc_ref[...] = acc_ref[...] * corr + pe
    m_ref[...] = m_new

    @pl.when(s == ns - 1)
    def _stash():
        pall_ref[pl.ds(N_HEADS * b, N_HEADS), :] = acc_ref[...] / l_ref[...]

    @pl.when((b == n_batch - 1) & (s == ns - 1))
    def _finalize():
        BH = n_batch * N_HEADS
        proj = jax.lax.dot_general(pall_ref[...], wv_ref[...],
                                   (((1,), (0,)), ((), ())),
                                   preferred_element_type=jnp.float32)
        row = jax.lax.broadcasted_iota(jnp.int32, (BH, D_MODEL), 0)
        col = jax.lax.broadcasted_iota(jnp.int32, (BH, D_MODEL), 1)
        mask = (col // DPH == row % N_HEADS).astype(jnp.float32)
        zall = (jnp.sum((proj * mask).reshape(n_batch, N_HEADS, D_MODEL), axis=1)
                + bv_ref[...])

        iota_k = jax.lax.broadcasted_iota(jnp.int32, (n_batch, K_CODES), 1)
        ssq = jnp.float32(0.0)
        idx_cols = []
        for h in range(QH):
            cbh = cb_ref[h]
            zh = zall[:, h * DPQ:(h + 1) * DPQ]
            dots = jax.lax.dot_general(zh, cbh, (((1,), (1,)), ((), ())),
                                       preferred_element_type=jnp.float32)
            csq = jax.lax.dot_general(jnp.ones((1, DPQ), jnp.float32), cbh * cbh,
                                      (((1,), (1,)), ((), ())),
                                      preferred_element_type=jnp.float32)
            zsq = jnp.sum(zh * zh, axis=1, keepdims=True)
            dist = zsq + csq - 2.0 * dots
            md = jnp.min(dist, axis=1, keepdims=True)
            idxs = jnp.min(jnp.where(dist == md, iota_k, K_CODES),
                           axis=1, keepdims=True)
            idx_cols.append(idxs)
            onehot = (iota_k == idxs).astype(jnp.float32)
            q = jax.lax.dot_general(onehot, cbh, (((1,), (0,)), ((), ())),
                                    preferred_element_type=jnp.float32)
            out_ref[:, 0, h * DPQ:(h + 1) * DPQ] = q
            d = q - zh
            ssq = ssq + jnp.sum(d * d)

        idx_ref[...] = jnp.concatenate(idx_cols, axis=1)
        loss_ref[0, 0] = ssq * loss_scale


def kernel(encoding, W_k, b_k, W_v, b_v, codebook, global_step):
    del b_k, global_step
    B, S, D = encoding.shape
    ns = S // S_BLK
    bv = b_v.reshape(1, D)
    body = functools.partial(_fused, loss_scale=0.25 / (B * QH * DPQ), n_batch=B)
    out, idx, loss = pl.pallas_call(
        body,
        grid=(B, ns),
        in_specs=[
            pl.BlockSpec((1, S_BLK, D), lambda b, s: (b, s, 0)),
            pl.BlockSpec((D, N_HEADS), lambda b, s: (0, 0)),
            pl.BlockSpec((D, D), lambda b, s: (0, 0)),
            pl.BlockSpec((1, D), lambda b, s: (0, 0)),
            pl.BlockSpec((QH, K_CODES, DPQ), lambda b, s: (0, 0, 0)),
        ],
        out_specs=[
            pl.BlockSpec((B, 1, D), lambda b, s: (0, 0, 0)),
            pl.BlockSpec((B, QH), lambda b, s: (0, 0)),
            pl.BlockSpec(memory_space=pltpu.SMEM),
        ],
        out_shape=[
            jax.ShapeDtypeStruct((B, 1, D), jnp.float32),
            jax.ShapeDtypeStruct((B, QH), jnp.int32),
            jax.ShapeDtypeStruct((1, 1), jnp.float32),
        ],
        scratch_shapes=[
            pltpu.VMEM((N_HEADS, 1), jnp.float32),
            pltpu.VMEM((N_HEADS, 1), jnp.float32),
            pltpu.VMEM((N_HEADS, D), jnp.float32),
            pltpu.VMEM((B * N_HEADS, D), jnp.float32),
        ],
    )(encoding, W_k, W_v, bv, codebook)
    return out, loss.reshape(()), idx


# speedup vs baseline: 1.0674x; 1.0417x over previous
import functools

import jax
import jax.numpy as jnp
from jax.experimental import pallas as pl
from jax.experimental.pallas import tpu as pltpu

D_MODEL = 1024
N_HEADS = 16
DPH = D_MODEL // N_HEADS
QH = 4
DPQ = D_MODEL // QH
K_CODES = 1024
S_BLK = 4096


def _fused(enc_ref, wk_ref, wv_ref, bv_ref, cb_ref,
           out_ref, idx_ref, loss_ref,
           m_ref, l_ref, acc_ref, pall_ref, *, loss_scale, n_batch):
    b = pl.program_id(0)
    s = pl.program_id(1)
    ns = pl.num_programs(1)

    @pl.when(s == 0)
    def _init():
        m_ref[...] = jnp.full_like(m_ref, -jnp.inf)
        l_ref[...] = jnp.zeros_like(l_ref)
        acc_ref[...] = jnp.zeros_like(acc_ref)

    enc = enc_ref[0]
    st = jax.lax.dot_general(wk_ref[...], enc, (((0,), (1,)), ((), ())),
                             preferred_element_type=jnp.float32)
    m_old = m_ref[...]
    m_new = jnp.maximum(m_old, jnp.max(st, axis=1, keepdims=True))
    corr = jnp.exp(m_old - m_new)
    p = jnp.exp(st - m_new)
    l_ref[...] = l_ref[...] * corr + jnp.sum(p, axis=1, keepdims=True)
    pe = jax.lax.dot_general(p, enc, (((1,), (0,)), ((), ())),
                             preferred_element_type=jnp.float32)
    acc_ref[...] = acc_ref[...] * corr + pe
    m_ref[...] = m_new

    @pl.when(s == ns - 1)
    def _stash():
        pall_ref[pl.ds(N_HEADS * b, N_HEADS), :] = acc_ref[...] / l_ref[...]

    @pl.when((b == n_batch - 1) & (s == ns - 1))
    def _finalize():
        BH = n_batch * N_HEADS
        proj = jax.lax.dot_general(pall_ref[...], wv_ref[...],
                                   (((1,), (0,)), ((), ())),
                                   preferred_element_type=jnp.float32)
        row = jax.lax.broadcasted_iota(jnp.int32, (BH, D_MODEL), 0)
        col = jax.lax.broadcasted_iota(jnp.int32, (BH, D_MODEL), 1)
        mask = (col // DPH == row % N_HEADS).astype(jnp.float32)
        zall = (jnp.sum((proj * mask).reshape(n_batch, N_HEADS, D_MODEL), axis=1)
                + bv_ref[...])

        iota_k = jax.lax.broadcasted_iota(jnp.int32, (n_batch, K_CODES), 1)
        dists = []
        for h in range(QH):
            cbh = cb_ref[h]
            zh = zall[:, h * DPQ:(h + 1) * DPQ]
            dots = jax.lax.dot_general(zh, cbh, (((1,), (1,)), ((), ())),
                                       preferred_element_type=jnp.float32)
            csq = jax.lax.dot_general(jnp.ones((1, DPQ), jnp.float32), cbh * cbh,
                                      (((1,), (1,)), ((), ())),
                                      preferred_element_type=jnp.float32)
            zsq = jnp.sum(zh * zh, axis=1, keepdims=True)
            dists.append(zsq + csq - 2.0 * dots)

        idx_cols = []
        for h in range(QH):
            md = jnp.min(dists[h], axis=1, keepdims=True)
            idx_cols.append(jnp.min(jnp.where(dists[h] == md, iota_k, K_CODES),
                                    axis=1, keepdims=True))

        ssq = jnp.float32(0.0)
        for h in range(QH):
            onehot = (iota_k == idx_cols[h]).astype(jnp.float32)
            q = jax.lax.dot_general(onehot, cb_ref[h], (((1,), (0,)), ((), ())),
                                    preferred_element_type=jnp.float32)
            out_ref[:, 0, h * DPQ:(h + 1) * DPQ] = q
            d = q - zall[:, h * DPQ:(h + 1) * DPQ]
            ssq = ssq + jnp.sum(d * d)

        idx_ref[...] = jnp.concatenate(idx_cols, axis=1)
        loss_ref[0, 0] = ssq * loss_scale


def kernel(encoding, W_k, b_k, W_v, b_v, codebook, global_step):
    del b_k, global_step
    B, S, D = encoding.shape
    ns = S // S_BLK
    bv = b_v.reshape(1, D)
    body = functools.partial(_fused, loss_scale=0.25 / (B * QH * DPQ), n_batch=B)
    out, idx, loss = pl.pallas_call(
        body,
        grid=(B, ns),
        in_specs=[
            pl.BlockSpec((1, S_BLK, D), lambda b, s: (b, s, 0)),
            pl.BlockSpec((D, N_HEADS), lambda b, s: (0, 0)),
            pl.BlockSpec((D, D), lambda b, s: (0, 0)),
            pl.BlockSpec((1, D), lambda b, s: (0, 0)),
            pl.BlockSpec((QH, K_CODES, DPQ), lambda b, s: (0, 0, 0)),
        ],
        out_specs=[
            pl.BlockSpec((B, 1, D), lambda b, s: (0, 0, 0)),
            pl.BlockSpec((B, QH), lambda b, s: (0, 0)),
            pl.BlockSpec(memory_space=pltpu.SMEM),
        ],
        out_shape=[
            jax.ShapeDtypeStruct((B, 1, D), jnp.float32),
            jax.ShapeDtypeStruct((B, QH), jnp.int32),
            jax.ShapeDtypeStruct((1, 1), jnp.float32),
        ],
        scratch_shapes=[
            pltpu.VMEM((N_HEADS, 1), jnp.float32),
            pltpu.VMEM((N_HEADS, 1), jnp.float32),
            pltpu.VMEM((N_HEADS, D), jnp.float32),
            pltpu.VMEM((B * N_HEADS, D), jnp.float32),
        ],
    )(encoding, W_k, W_v, bv, codebook)
    return out, loss.reshape(()), idx
